# baseline (speedup 1.0000x reference)
import functools

import jax
import jax.numpy as jnp
from jax import lax
from jax.experimental import pallas as pl
from jax.experimental.pallas import tpu as pltpu
from jax.experimental.pallas import tpu_sc as plsc

H = 768
E = 64
K = 2
TM = 4096
NW = 32
CHUNK = 256
L = 16


def _mlp_body(x_ref, w1_ref, b1_ref, w2_ref, b2_ref, e_ref):
    x = x_ref[0]
    h = jnp.dot(x, w1_ref[...], preferred_element_type=jnp.float32)
    h = h + b1_ref[...]
    h = 0.5 * h * (1.0 + jax.lax.erf(h * 0.7071067811865476))
    e = jnp.dot(h, w2_ref[...], preferred_element_type=jnp.float32)
    e_ref[0] = e + b2_ref[...]


def _sc_route_body(N, ew_hbm, masks_hbm, usage_hbm, cnt_hbm,
                   ew_v, masks_v, usage_v, cnt_v):
    wid = lax.axis_index("c") * 16 + lax.axis_index("s")
    tpw = N // NW
    lanes = lax.iota(jnp.int32, L)
    zero16 = jnp.zeros((L,), jnp.float32)
    ninf16 = jnp.full((L,), -jnp.inf, jnp.float32)
    izero16 = jnp.zeros((L,), jnp.int32)

    u = [zero16] * 4
    cnt = zero16

    for c in range(tpw // CHUNK):
        base = (wid * tpw + c * CHUNK) * E
        pltpu.sync_copy(ew_hbm.at[pl.ds(base, CHUNK * E)], ew_v)

        def group_body(g, cnt):
            rowbase = (lanes + g * L) * E

            def estep(e, carry):
                m1, m2, i1, i2 = carry
                col = jnp.full((L,), e, jnp.int32)
                v = plsc.load_gather(ew_v, [rowbase + e])
                plsc.store_scatter(masks_v, [rowbase + e], zero16)
                gt1 = v > m1
                gt2 = v > m2
                i2n = jnp.where(gt1, i1, jnp.where(gt2, col, i2))
                m2n = jnp.where(gt1, m1, jnp.where(gt2, v, m2))
                i1n = jnp.where(gt1, col, i1)
                m1n = jnp.where(gt1, v, m1)
                return (m1n, m2n, i1n, i2n)

            m1, m2, i1, i2 = lax.fori_loop(
                0, E, estep, (ninf16, ninf16, izero16, izero16))
            eps = jnp.exp(m2 - m1)
            denom = 1.0 + eps
            w1 = 1.0 / denom
            w2 = eps / denom
            plsc.store_scatter(masks_v, [rowbase + i1], w1)
            plsc.store_scatter(masks_v, [rowbase + i2], w2)
            return cnt + 1.0 + (w2 > 0.0).astype(jnp.float32)

        cnt = lax.fori_loop(0, CHUNK // L, group_body, cnt)

        def usage_body(t, u):
            tb = t * E
            return tuple(
                u[j] + plsc.load_gather(masks_v, [lanes + (tb + L * j)])
                for j in range(4))

        u = list(lax.fori_loop(0, CHUNK, usage_body, tuple(u)))
        pltpu.sync_copy(masks_v, masks_hbm.at[pl.ds(base, CHUNK * E)])

    for j in range(4):
        usage_v[pl.ds(L * j, L)] = u[j]
    cnt_v[...] = cnt
    pltpu.sync_copy(usage_v, usage_hbm.at[pl.ds(wid * E, E)])
    pltpu.sync_copy(cnt_v, cnt_hbm.at[pl.ds(wid * L, L)])


def _combine_body(ntokens, up_ref, cnt_ref, u_ref, l_ref):
    u = jnp.sum(up_ref[...], axis=0, keepdims=True)
    un = u / jnp.sum(u)
    u_ref[...] = un
    lbl = jnp.mean((un - 1.0 / E) ** 2)
    sparsity = jnp.sum(cnt_ref[...]) / (ntokens * K)
    l_ref[...] = jnp.reshape(lbl + 0.1 * sparsity, (1, 1))


def kernel(x, W1, b1, W2, b2):
    B, S, _ = x.shape
    N = B * S
    s_steps = S // TM

    e_out = pl.pallas_call(
        _mlp_body,
        grid=(B, s_steps),
        in_specs=[
            pl.BlockSpec((1, TM, H), lambda b, s: (b, s, 0)),
            pl.BlockSpec((H, 2 * H), lambda b, s: (0, 0)),
            pl.BlockSpec((1, 2 * H), lambda b, s: (0, 0)),
            pl.BlockSpec((2 * H, E), lambda b, s: (0, 0)),
            pl.BlockSpec((1, E), lambda b, s: (0, 0)),
        ],
        out_specs=pl.BlockSpec((1, TM, E), lambda b, s: (b, s, 0)),
        out_shape=jax.ShapeDtypeStruct((B, S, E), jnp.float32),
        compiler_params=pltpu.CompilerParams(
            dimension_semantics=("arbitrary", "arbitrary"),
        ),
    )(x, W1, b1.reshape(1, 2 * H), W2, b2.reshape(1, E))

    mesh = plsc.VectorSubcoreMesh(core_axis_name="c", subcore_axis_name="s")
    sc_route = functools.partial(
        pl.kernel,
        out_type=[
            jax.ShapeDtypeStruct((N * E,), jnp.float32),
            jax.ShapeDtypeStruct((NW * E,), jnp.float32),
            jax.ShapeDtypeStruct((NW * L,), jnp.float32),
        ],
        mesh=mesh,
        scratch_types=[
            pltpu.VMEM((CHUNK * E,), jnp.float32),
            pltpu.VMEM((CHUNK * E,), jnp.float32),
            pltpu.VMEM((E,), jnp.float32),
            pltpu.VMEM((L,), jnp.float32),
        ],
        compiler_params=pltpu.CompilerParams(
            use_tc_tiling_on_sc=False, needs_layout_passes=False),
    )(functools.partial(_sc_route_body, N))
    masks_flat, usage_p, cnt_p = sc_route(e_out.reshape(N * E))
    masks = masks_flat.reshape(B, S, E)

    usage, loss = pl.pallas_call(
        functools.partial(_combine_body, N),
        grid=(1,),
        in_specs=[
            pl.BlockSpec((NW, E), lambda i: (0, 0)),
            pl.BlockSpec((NW, L), lambda i: (0, 0)),
        ],
        out_specs=[
            pl.BlockSpec((1, E), lambda i: (0, 0)),
            pl.BlockSpec((1, 1), lambda i: (0, 0)),
        ],
        out_shape=[
            jax.ShapeDtypeStruct((1, E), jnp.float32),
            jax.ShapeDtypeStruct((1, 1), jnp.float32),
        ],
    )(usage_p.reshape(NW, E), cnt_p.reshape(NW, L))

    return (e_out, masks, loss[0, 0], usage[0])


# speedup vs baseline: 1.0745x; 1.0745x over previous
import functools

import jax
import jax.numpy as jnp
from jax import lax
from jax.experimental import pallas as pl
from jax.experimental.pallas import tpu as pltpu
from jax.experimental.pallas import tpu_sc as plsc

H = 768
E = 64
K = 2
TM = 4096
NW = 32
CHUNK = 512
L = 16


def _mlp_body(x_ref, w1_ref, b1_ref, w2_ref, b2_ref, e_ref):
    x = x_ref[0]
    h = jnp.dot(x, w1_ref[...], preferred_element_type=jnp.float32)
    h = h + b1_ref[...]
    h = 0.5 * h * (1.0 + jax.lax.erf(h * 0.7071067811865476))
    e = jnp.dot(h, w2_ref[...], preferred_element_type=jnp.float32)
    e_ref[0] = e + b2_ref[...]


def _sc_route_body(N, ew_hbm, masks_hbm, usage_hbm, cnt_hbm,
                   ew_v, masks_v, usage_v, cnt_v):
    wid = lax.axis_index("c") * 16 + lax.axis_index("s")
    tpw = N // NW
    lanes = lax.iota(jnp.int32, L)
    zero16 = jnp.zeros((L,), jnp.float32)
    ninf16 = jnp.full((L,), -jnp.inf, jnp.float32)
    izero16 = jnp.zeros((L,), jnp.int32)

    cnt = zero16
    for j in range(4):
        usage_v[pl.ds(L * j, L)] = zero16

    for c in range(tpw // CHUNK):
        base = (wid * tpw + c * CHUNK) * E
        pltpu.sync_copy(ew_hbm.at[pl.ds(base, CHUNK * E)], ew_v)

        def group_body(g, cnt):
            rowbase = (lanes + g * L) * E

            def estep(eo, carry):
                m1, m2, i1, i2 = carry
                for k in range(8):
                    e = eo * 8 + k
                    col = jnp.full((L,), e, jnp.int32)
                    v = plsc.load_gather(ew_v, [rowbase + e])
                    plsc.store_scatter(masks_v, [rowbase + e], zero16)
                    gt1 = v > m1
                    gt2 = v > m2
                    i2 = jnp.where(gt1, i1, jnp.where(gt2, col, i2))
                    m2 = jnp.maximum(m2, jnp.minimum(v, m1))
                    i1 = jnp.where(gt1, col, i1)
                    m1 = jnp.maximum(m1, v)
                return (m1, m2, i1, i2)

            m1, m2, i1, i2 = lax.fori_loop(
                0, E // 8, estep, (ninf16, ninf16, izero16, izero16))
            eps = jnp.exp(m2 - m1)
            denom = 1.0 + eps
            w1 = 1.0 / denom
            w2 = eps / denom
            plsc.store_scatter(masks_v, [rowbase + i1], w1)
            plsc.store_scatter(masks_v, [rowbase + i2], w2)
            plsc.addupdate_scatter(usage_v, [i1], w1)
            plsc.addupdate_scatter(usage_v, [i2], w2)
            return cnt + 1.0 + (w2 > 0.0).astype(jnp.float32)

        cnt = lax.fori_loop(0, CHUNK // L, group_body, cnt)
        pltpu.sync_copy(masks_v, masks_hbm.at[pl.ds(base, CHUNK * E)])

    cnt_v[...] = cnt
    pltpu.sync_copy(usage_v, usage_hbm.at[pl.ds(wid * E, E)])
    pltpu.sync_copy(cnt_v, cnt_hbm.at[pl.ds(wid * L, L)])


def _combine_body(ntokens, up_ref, cnt_ref, u_ref, l_ref):
    u = jnp.sum(up_ref[...], axis=0, keepdims=True)
    un = u / jnp.sum(u)
    u_ref[...] = un
    lbl = jnp.mean((un - 1.0 / E) ** 2)
    sparsity = jnp.sum(cnt_ref[...]) / (ntokens * K)
    l_ref[...] = jnp.reshape(lbl + 0.1 * sparsity, (1, 1))


def kernel(x, W1, b1, W2, b2):
    B, S, _ = x.shape
    N = B * S
    s_steps = S // TM

    e_out = pl.pallas_call(
        _mlp_body,
        grid=(B, s_steps),
        in_specs=[
            pl.BlockSpec((1, TM, H), lambda b, s: (b, s, 0)),
            pl.BlockSpec((H, 2 * H), lambda b, s: (0, 0)),
            pl.BlockSpec((1, 2 * H), lambda b, s: (0, 0)),
            pl.BlockSpec((2 * H, E), lambda b, s: (0, 0)),
            pl.BlockSpec((1, E), lambda b, s: (0, 0)),
        ],
        out_specs=pl.BlockSpec((1, TM, E), lambda b, s: (b, s, 0)),
        out_shape=jax.ShapeDtypeStruct((B, S, E), jnp.float32),
        compiler_params=pltpu.CompilerParams(
            dimension_semantics=("arbitrary", "arbitrary"),
        ),
    )(x, W1, b1.reshape(1, 2 * H), W2, b2.reshape(1, E))

    mesh = plsc.VectorSubcoreMesh(core_axis_name="c", subcore_axis_name="s")
    sc_route = functools.partial(
        pl.kernel,
        out_type=[
            jax.ShapeDtypeStruct((N * E,), jnp.float32),
            jax.ShapeDtypeStruct((NW * E,), jnp.float32),
            jax.ShapeDtypeStruct((NW * L,), jnp.float32),
        ],
        mesh=mesh,
        scratch_types=[
            pltpu.VMEM((CHUNK * E,), jnp.float32),
            pltpu.VMEM((CHUNK * E,), jnp.float32),
            pltpu.VMEM((E,), jnp.float32),
            pltpu.VMEM((L,), jnp.float32),
        ],
        compiler_params=pltpu.CompilerParams(
            use_tc_tiling_on_sc=False, needs_layout_passes=False),
    )(functools.partial(_sc_route_body, N))
    masks_flat, usage_p, cnt_p = sc_route(e_out.reshape(N * E))
    masks = masks_flat.reshape(B, S, E)

    usage, loss = pl.pallas_call(
        functools.partial(_combine_body, N),
        grid=(1,),
        in_specs=[
            pl.BlockSpec((NW, E), lambda i: (0, 0)),
            pl.BlockSpec((NW, L), lambda i: (0, 0)),
        ],
        out_specs=[
            pl.BlockSpec((1, E), lambda i: (0, 0)),
            pl.BlockSpec((1, 1), lambda i: (0, 0)),
        ],
        out_shape=[
            jax.ShapeDtypeStruct((1, E), jnp.float32),
            jax.ShapeDtypeStruct((1, 1), jnp.float32),
        ],
    )(usage_p.reshape(NW, E), cnt_p.reshape(NW, L))

    return (e_out, masks, loss[0, 0], usage[0])


# speedup vs baseline: 1.0770x; 1.0023x over previous
import functools

import jax
import jax.numpy as jnp
from jax import lax
from jax.experimental import pallas as pl
from jax.experimental.pallas import tpu as pltpu
from jax.experimental.pallas import tpu_sc as plsc

H = 768
E = 64
K = 2
TM = 4096
NW = 32
CHUNK = 512
L = 16


def _mlp_body(x_ref, w1_ref, b1_ref, w2_ref, b2_ref, e_ref):
    x = x_ref[0]
    h = jnp.dot(x, w1_ref[...], preferred_element_type=jnp.float32)
    h = h + b1_ref[...]
    h = 0.5 * h * (1.0 + jax.lax.erf(h * 0.7071067811865476))
    e = jnp.dot(h, w2_ref[...], preferred_element_type=jnp.float32)
    e_ref[0] = e + b2_ref[...]


def _sc_route_body(N, ew_hbm, masks_hbm, usage_hbm, cnt_hbm,
                   ew_v, masks_v, usage_v, cnt_v):
    wid = lax.axis_index("c") * 16 + lax.axis_index("s")
    tpw = N // NW
    lanes = lax.iota(jnp.int32, L)
    zero16 = jnp.zeros((L,), jnp.float32)
    ninf16 = jnp.full((L,), -jnp.inf, jnp.float32)
    izero16 = jnp.zeros((L,), jnp.int32)

    cnt = zero16
    for j in range(4):
        usage_v[pl.ds(L * j, L)] = zero16

    for c in range(tpw // CHUNK):
        base = (wid * tpw + c * CHUNK) * E
        pltpu.sync_copy(ew_hbm.at[pl.ds(base, CHUNK * E)], ew_v)

        def group_body(g, cnt):
            rbs = [(lanes + (2 * g + p) * L) * E for p in range(2)]

            def estep(eo, carry):
                st = list(carry)
                for k in range(4):
                    e = eo * 4 + k
                    col = jnp.full((L,), e, jnp.int32)
                    for p in range(2):
                        m1, m2, i1, i2 = st[p]
                        v = plsc.load_gather(ew_v, [rbs[p] + e])
                        plsc.store_scatter(masks_v, [rbs[p] + e], zero16)
                        gt1 = v > m1
                        gt2 = v > m2
                        i2 = jnp.where(gt1, i1, jnp.where(gt2, col, i2))
                        m2 = jnp.maximum(m2, jnp.minimum(v, m1))
                        i1 = jnp.where(gt1, col, i1)
                        m1 = jnp.maximum(m1, v)
                        st[p] = (m1, m2, i1, i2)
                return tuple(st)

            init = ((ninf16, ninf16, izero16, izero16),) * 2
            st = lax.fori_loop(0, E // 4, estep, init)
            for p in range(2):
                m1, m2, i1, i2 = st[p]
                eps = jnp.exp(m2 - m1)
                denom = 1.0 + eps
                w1 = 1.0 / denom
                w2 = eps / denom
                plsc.store_scatter(masks_v, [rbs[p] + i1], w1)
                plsc.store_scatter(masks_v, [rbs[p] + i2], w2)
                plsc.addupdate_scatter(usage_v, [i1], w1)
                plsc.addupdate_scatter(usage_v, [i2], w2)
                cnt = cnt + 1.0 + (w2 > 0.0).astype(jnp.float32)
            return cnt

        cnt = lax.fori_loop(0, CHUNK // L // 2, group_body, cnt)
        pltpu.sync_copy(masks_v, masks_hbm.at[pl.ds(base, CHUNK * E)])

    cnt_v[...] = cnt
    pltpu.sync_copy(usage_v, usage_hbm.at[pl.ds(wid * E, E)])
    pltpu.sync_copy(cnt_v, cnt_hbm.at[pl.ds(wid * L, L)])


def _combine_body(ntokens, up_ref, cnt_ref, u_ref, l_ref):
    u = jnp.sum(up_ref[...], axis=0, keepdims=True)
    un = u / jnp.sum(u)
    u_ref[...] = un
    lbl = jnp.mean((un - 1.0 / E) ** 2)
    sparsity = jnp.sum(cnt_ref[...]) / (ntokens * K)
    l_ref[...] = jnp.reshape(lbl + 0.1 * sparsity, (1, 1))


def kernel(x, W1, b1, W2, b2):
    B, S, _ = x.shape
    N = B * S
    s_steps = S // TM

    e_out = pl.pallas_call(
        _mlp_body,
        grid=(B, s_steps),
        in_specs=[
            pl.BlockSpec((1, TM, H), lambda b, s: (b, s, 0)),
            pl.BlockSpec((H, 2 * H), lambda b, s: (0, 0)),
            pl.BlockSpec((1, 2 * H), lambda b, s: (0, 0)),
            pl.BlockSpec((2 * H, E), lambda b, s: (0, 0)),
            pl.BlockSpec((1, E), lambda b, s: (0, 0)),
        ],
        out_specs=pl.BlockSpec((1, TM, E), lambda b, s: (b, s, 0)),
        out_shape=jax.ShapeDtypeStruct((B, S, E), jnp.float32),
        compiler_params=pltpu.CompilerParams(
            dimension_semantics=("arbitrary", "arbitrary"),
        ),
    )(x, W1, b1.reshape(1, 2 * H), W2, b2.reshape(1, E))

    mesh = plsc.VectorSubcoreMesh(core_axis_name="c", subcore_axis_name="s")
    sc_route = functools.partial(
        pl.kernel,
        out_type=[
            jax.ShapeDtypeStruct((N * E,), jnp.float32),
            jax.ShapeDtypeStruct((NW * E,), jnp.float32),
            jax.ShapeDtypeStruct((NW * L,), jnp.float32),
        ],
        mesh=mesh,
        scratch_types=[
            pltpu.VMEM((CHUNK * E,), jnp.float32),
            pltpu.VMEM((CHUNK * E,), jnp.float32),
            pltpu.VMEM((E,), jnp.float32),
            pltpu.VMEM((L,), jnp.float32),
        ],
        compiler_params=pltpu.CompilerParams(
            use_tc_tiling_on_sc=False, needs_layout_passes=False),
    )(functools.partial(_sc_route_body, N))
    masks_flat, usage_p, cnt_p = sc_route(e_out.reshape(N * E))
    masks = masks_flat.reshape(B, S, E)

    usage, loss = pl.pallas_call(
        functools.partial(_combine_body, N),
        grid=(1,),
        in_specs=[
            pl.BlockSpec((NW, E), lambda i: (0, 0)),
            pl.BlockSpec((NW, L), lambda i: (0, 0)),
        ],
        out_specs=[
            pl.BlockSpec((1, E), lambda i: (0, 0)),
            pl.BlockSpec((1, 1), lambda i: (0, 0)),
        ],
        out_shape=[
            jax.ShapeDtypeStruct((1, E), jnp.float32),
            jax.ShapeDtypeStruct((1, 1), jnp.float32),
        ],
    )(usage_p.reshape(NW, E), cnt_p.reshape(NW, L))

    return (e_out, masks, loss[0, 0], usage[0])


# speedup vs baseline: 1.1516x; 1.0692x over previous
import functools

import jax
import jax.numpy as jnp
from jax import lax
from jax.experimental import pallas as pl
from jax.experimental.pallas import tpu as pltpu
from jax.experimental.pallas import tpu_sc as plsc

H = 768
E = 64
K = 2
TM = 4096
NW = 32
CHUNK = 256
L = 16


def _mlp_body(x_ref, w1_ref, b1_ref, w2_ref, b2_ref, e_ref):
    x = x_ref[0]
    h = jnp.dot(x, w1_ref[...], preferred_element_type=jnp.float32)
    h = h + b1_ref[...]
    h = 0.5 * h * (1.0 + jax.lax.erf(h * 0.7071067811865476))
    e = jnp.dot(h, w2_ref[...], preferred_element_type=jnp.float32)
    e_ref[0] = e + b2_ref[...]


def _sc_route_body(N, ew_hbm, masks_hbm, usage_hbm, cnt_hbm,
                   ew_v, masks_v, usage_v, cnt_v):
    wid = lax.axis_index("c") * 16 + lax.axis_index("s")
    tpw = N // NW
    lanes = lax.iota(jnp.int32, L)
    zero16 = jnp.zeros((L,), jnp.float32)
    ninf16 = jnp.full((L,), -jnp.inf, jnp.float32)
    izero16 = jnp.zeros((L,), jnp.int32)

    cnt = zero16
    for j in range(4):
        usage_v[pl.ds(L * j, L)] = zero16

    for c in range(tpw // CHUNK):
        base = (wid * tpw + c * CHUNK) * E
        pltpu.sync_copy(ew_hbm.at[pl.ds(base, CHUNK * E)], ew_v)

        def group_body(g, cnt):
            rbs = [(lanes + (2 * g + p) * L) * E for p in range(2)]

            def estep(eo, carry):
                st = list(carry)
                for k in range(4):
                    e = eo * 4 + k
                    col = jnp.full((L,), e, jnp.int32)
                    for p in range(2):
                        m1, m2, i1, i2 = st[p]
                        v = plsc.load_gather(ew_v, [rbs[p] + e])
                        plsc.store_scatter(masks_v, [rbs[p] + e], zero16)
                        gt1 = v > m1
                        gt2 = v > m2
                        i2 = jnp.where(gt1, i1, jnp.where(gt2, col, i2))
                        m2 = jnp.maximum(m2, jnp.minimum(v, m1))
                        i1 = jnp.where(gt1, col, i1)
                        m1 = jnp.maximum(m1, v)
                        st[p] = (m1, m2, i1, i2)
                return tuple(st)

            init = ((ninf16, ninf16, izero16, izero16),) * 2
            st = lax.fori_loop(0, E // 4, estep, init)
            for p in range(2):
                m1, m2, i1, i2 = st[p]
                eps = jnp.exp(m2 - m1)
                denom = 1.0 + eps
                w1 = 1.0 / denom
                w2 = eps / denom
                plsc.store_scatter(masks_v, [rbs[p] + i1], w1)
                plsc.store_scatter(masks_v, [rbs[p] + i2], w2)
                plsc.addupdate_scatter(usage_v, [i1], w1)
                plsc.addupdate_scatter(usage_v, [i2], w2)
                cnt = cnt + 1.0 + (w2 > 0.0).astype(jnp.float32)
            return cnt

        cnt = lax.fori_loop(0, CHUNK // L // 2, group_body, cnt)
        pltpu.sync_copy(masks_v, masks_hbm.at[pl.ds(base, CHUNK * E)])

    cnt_v[...] = cnt
    pltpu.sync_copy(usage_v, usage_hbm.at[pl.ds(wid * E, E)])
    pltpu.sync_copy(cnt_v, cnt_hbm.at[pl.ds(wid * L, L)])


def _combine_body(ntokens, nchunks, *refs):
    up_refs = refs[:nchunks]
    cnt_refs = refs[nchunks:2 * nchunks]
    u_ref, l_ref = refs[2 * nchunks], refs[2 * nchunks + 1]
    u = sum(jnp.sum(r[...], axis=0, keepdims=True) for r in up_refs)
    un = u / jnp.sum(u)
    u_ref[...] = un
    lbl = jnp.mean((un - 1.0 / E) ** 2)
    sparsity = sum(jnp.sum(r[...]) for r in cnt_refs) / (ntokens * K)
    l_ref[...] = jnp.reshape(lbl + 0.1 * sparsity, (1, 1))


def kernel(x, W1, b1, W2, b2):
    B, S, _ = x.shape
    N = B * S
    s_steps = S // TM
    b1r = b1.reshape(1, 2 * H)
    b2r = b2.reshape(1, E)

    mesh = plsc.VectorSubcoreMesh(core_axis_name="c", subcore_axis_name="s")
    sc_route = functools.partial(
        pl.kernel,
        out_type=[
            jax.ShapeDtypeStruct((S * E,), jnp.float32),
            jax.ShapeDtypeStruct((NW * E,), jnp.float32),
            jax.ShapeDtypeStruct((NW * L,), jnp.float32),
        ],
        mesh=mesh,
        scratch_types=[
            pltpu.VMEM((CHUNK * E,), jnp.float32),
            pltpu.VMEM((CHUNK * E,), jnp.float32),
            pltpu.VMEM((E,), jnp.float32),
            pltpu.VMEM((L,), jnp.float32),
        ],
        compiler_params=pltpu.CompilerParams(
            use_tc_tiling_on_sc=False, needs_layout_passes=False),
    )(functools.partial(_sc_route_body, S))

    ews, mflats, usage_ps, cnt_ps = [], [], [], []
    for b in range(B):
        ew_b = pl.pallas_call(
            _mlp_body,
            grid=(1, s_steps),
            in_specs=[
                pl.BlockSpec((1, TM, H), lambda _, s, b=b: (b, s, 0)),
                pl.BlockSpec((H, 2 * H), lambda _, s: (0, 0)),
                pl.BlockSpec((1, 2 * H), lambda _, s: (0, 0)),
                pl.BlockSpec((2 * H, E), lambda _, s: (0, 0)),
                pl.BlockSpec((1, E), lambda _, s: (0, 0)),
            ],
            out_specs=pl.BlockSpec((1, TM, E), lambda _, s: (0, s, 0)),
            out_shape=jax.ShapeDtypeStruct((1, S, E), jnp.float32),
            compiler_params=pltpu.CompilerParams(
                dimension_semantics=("arbitrary", "arbitrary"),
            ),
        )(x, W1, b1r, W2, b2r)
        ews.append(ew_b)
        mf, up, cp = sc_route(ew_b.reshape(S * E))
        mflats.append(mf)
        usage_ps.append(up)
        cnt_ps.append(cp)

    e_out = jnp.concatenate(ews, axis=0)
    masks = jnp.concatenate(mflats).reshape(B, S, E)

    usage, loss = pl.pallas_call(
        functools.partial(_combine_body, N, B),
        grid=(1,),
        in_specs=(
            [pl.BlockSpec((NW, E), lambda i: (0, 0))] * B
            + [pl.BlockSpec((NW, L), lambda i: (0, 0))] * B
        ),
        out_specs=[
            pl.BlockSpec((1, E), lambda i: (0, 0)),
            pl.BlockSpec((1, 1), lambda i: (0, 0)),
        ],
        out_shape=[
            jax.ShapeDtypeStruct((1, E), jnp.float32),
            jax.ShapeDtypeStruct((1, 1), jnp.float32),
        ],
    )(*[u.reshape(NW, E) for u in usage_ps],
      *[c.reshape(NW, L) for c in cnt_ps])

    return (e_out, masks, loss[0, 0], usage[0])


# speedup vs baseline: 1.1808x; 1.0254x over previous
import functools

import jax
import jax.numpy as jnp
from jax import lax
from jax.experimental import pallas as pl
from jax.experimental.pallas import tpu as pltpu
from jax.experimental.pallas import tpu_sc as plsc

H = 768
E = 64
K = 2
TM = 4096
NW = 32
CHUNK = 256
L = 16


def _mlp_body(x_ref, w1_ref, b1_ref, w2_ref, b2_ref, e_ref):
    x = x_ref[0]
    h = jnp.dot(x, w1_ref[...], preferred_element_type=jnp.float32)
    h = h + b1_ref[...]
    h = 0.5 * h * (1.0 + jax.lax.erf(h * 0.7071067811865476))
    e = jnp.dot(h, w2_ref[...], preferred_element_type=jnp.float32)
    e_ref[0] = e + b2_ref[...]


def _sc_route_body(N, ew_hbm, masks_hbm, usage_hbm, cnt_hbm,
                   ew_v, masks_v, usage_v, cnt_v):
    wid = lax.axis_index("c") * 16 + lax.axis_index("s")
    tpw = N // NW
    lanes = lax.iota(jnp.int32, L)
    zero16 = jnp.zeros((L,), jnp.float32)
    ninf16 = jnp.full((L,), -jnp.inf, jnp.float32)
    izero16 = jnp.zeros((L,), jnp.int32)

    cnt = zero16
    for j in range(4):
        usage_v[pl.ds(L * j, L)] = zero16

    for c in range(tpw // CHUNK):
        base = (wid * tpw + c * CHUNK) * E
        pltpu.sync_copy(ew_hbm.at[pl.ds(base, CHUNK * E)], ew_v)

        def group_body(g, cnt):
            rbs = [(lanes + (2 * g + p) * L) * E for p in range(2)]

            def estep(eo, carry):
                st = list(carry)
                for k in range(4):
                    e = eo * 4 + k
                    col = jnp.full((L,), e, jnp.int32)
                    for p in range(2):
                        m1, m2, i1, i2 = st[p]
                        v = plsc.load_gather(ew_v, [rbs[p] + e])
                        plsc.store_scatter(masks_v, [rbs[p] + e], zero16)
                        gt1 = v > m1
                        gt2 = v > m2
                        i2 = jnp.where(gt1, i1, jnp.where(gt2, col, i2))
                        m2 = jnp.maximum(m2, jnp.minimum(v, m1))
                        i1 = jnp.where(gt1, col, i1)
                        m1 = jnp.maximum(m1, v)
                        st[p] = (m1, m2, i1, i2)
                return tuple(st)

            init = ((ninf16, ninf16, izero16, izero16),) * 2
            st = lax.fori_loop(0, E // 4, estep, init)
            for p in range(2):
                m1, m2, i1, i2 = st[p]
                eps = jnp.exp(m2 - m1)
                denom = 1.0 + eps
                w1 = 1.0 / denom
                w2 = eps / denom
                plsc.store_scatter(masks_v, [rbs[p] + i1], w1)
                plsc.store_scatter(masks_v, [rbs[p] + i2], w2)
                plsc.addupdate_scatter(usage_v, [i1], w1)
                plsc.addupdate_scatter(usage_v, [i2], w2)
                cnt = cnt + 1.0 + (w2 > 0.0).astype(jnp.float32)
            return cnt

        cnt = lax.fori_loop(0, CHUNK // L // 2, group_body, cnt)
        pltpu.sync_copy(masks_v, masks_hbm.at[pl.ds(base, CHUNK * E)])

    cnt_v[...] = cnt
    pltpu.sync_copy(usage_v, usage_hbm.at[pl.ds(wid * E, E)])
    pltpu.sync_copy(cnt_v, cnt_hbm.at[pl.ds(wid * L, L)])


def _combine_body(ntokens, nchunks, *refs):
    up_refs = refs[:nchunks]
    cnt_refs = refs[nchunks:2 * nchunks]
    u_ref, l_ref = refs[2 * nchunks], refs[2 * nchunks + 1]
    u = sum(jnp.sum(r[...], axis=0, keepdims=True) for r in up_refs)
    un = u / jnp.sum(u)
    u_ref[...] = un
    lbl = jnp.mean((un - 1.0 / E) ** 2)
    sparsity = sum(jnp.sum(r[...]) for r in cnt_refs) / (ntokens * K)
    l_ref[...] = jnp.reshape(lbl + 0.1 * sparsity, (1, 1))


def kernel(x, W1, b1, W2, b2):
    B, S, _ = x.shape
    N = B * S
    s_steps = S // TM
    b1r = b1.reshape(1, 2 * H)
    b2r = b2.reshape(1, E)

    P = 2
    bs_per = B // P
    mesh = plsc.VectorSubcoreMesh(core_axis_name="c", subcore_axis_name="s")
    sc_route = functools.partial(
        pl.kernel,
        out_type=[
            jax.ShapeDtypeStruct((bs_per * S * E,), jnp.float32),
            jax.ShapeDtypeStruct((NW * E,), jnp.float32),
            jax.ShapeDtypeStruct((NW * L,), jnp.float32),
        ],
        mesh=mesh,
        scratch_types=[
            pltpu.VMEM((CHUNK * E,), jnp.float32),
            pltpu.VMEM((CHUNK * E,), jnp.float32),
            pltpu.VMEM((E,), jnp.float32),
            pltpu.VMEM((L,), jnp.float32),
        ],
        compiler_params=pltpu.CompilerParams(
            use_tc_tiling_on_sc=False, needs_layout_passes=False),
    )(functools.partial(_sc_route_body, bs_per * S))

    ews, mflats, usage_ps, cnt_ps = [], [], [], []
    for j in range(P):
        ew_j = pl.pallas_call(
            _mlp_body,
            grid=(bs_per, s_steps),
            in_specs=[
                pl.BlockSpec((1, TM, H),
                             lambda bb, s, j=j: (j * bs_per + bb, s, 0)),
                pl.BlockSpec((H, 2 * H), lambda bb, s: (0, 0)),
                pl.BlockSpec((1, 2 * H), lambda bb, s: (0, 0)),
                pl.BlockSpec((2 * H, E), lambda bb, s: (0, 0)),
                pl.BlockSpec((1, E), lambda bb, s: (0, 0)),
            ],
            out_specs=pl.BlockSpec((1, TM, E), lambda bb, s: (bb, s, 0)),
            out_shape=jax.ShapeDtypeStruct((bs_per, S, E), jnp.float32),
            compiler_params=pltpu.CompilerParams(
                dimension_semantics=("arbitrary", "arbitrary"),
            ),
        )(x, W1, b1r, W2, b2r)
        ews.append(ew_j)
        mf, up, cp = sc_route(ew_j.reshape(bs_per * S * E))
        mflats.append(mf)
        usage_ps.append(up)
        cnt_ps.append(cp)

    e_out = jnp.concatenate(ews, axis=0)
    masks = jnp.concatenate(mflats).reshape(B, S, E)

    usage, loss = pl.pallas_call(
        functools.partial(_combine_body, N, P),
        grid=(1,),
        in_specs=(
            [pl.BlockSpec((NW, E), lambda i: (0, 0))] * P
            + [pl.BlockSpec((NW, L), lambda i: (0, 0))] * P
        ),
        out_specs=[
            pl.BlockSpec((1, E), lambda i: (0, 0)),
            pl.BlockSpec((1, 1), lambda i: (0, 0)),
        ],
        out_shape=[
            jax.ShapeDtypeStruct((1, E), jnp.float32),
            jax.ShapeDtypeStruct((1, 1), jnp.float32),
        ],
    )(*[u.reshape(NW, E) for u in usage_ps],
      *[c.reshape(NW, L) for c in cnt_ps])

    return (e_out, masks, loss[0, 0], usage[0])
